# baseline (speedup 1.0000x reference)
import functools

import jax
import jax.numpy as jnp
import numpy as np
from jax.experimental import pallas as pl

_KS0 = np.uint32(0)
_KS1 = np.uint32(42)
_KS2 = np.uint32(0x1BD11BDA ^ 42)
_ROT0 = (13, 15, 26, 6)
_ROT1 = (17, 29, 16, 24)
_TINY = np.float32(np.finfo(np.float32).tiny)


def _rotl(x, d):
    return (x << np.uint32(d)) | (x >> np.uint32(32 - d))


def _threefry_bits(lo):
    x0 = jnp.zeros_like(lo) + _KS0
    x1 = lo + _KS1

    def rounds(x0, x1, rots):
        for r in rots:
            x0 = x0 + x1
            x1 = _rotl(x1, r)
            x1 = x1 ^ x0
        return x0, x1

    x0, x1 = rounds(x0, x1, _ROT0)
    x0 = x0 + _KS1
    x1 = x1 + np.uint32(_KS2 + np.uint32(1))
    x0, x1 = rounds(x0, x1, _ROT1)
    x0 = x0 + _KS2
    x1 = x1 + np.uint32(_KS0 + np.uint32(2))
    x0, x1 = rounds(x0, x1, _ROT0)
    x0 = x0 + _KS0
    x1 = x1 + np.uint32(_KS1 + np.uint32(3))
    x0, x1 = rounds(x0, x1, _ROT1)
    x0 = x0 + _KS1
    x1 = x1 + np.uint32(_KS2 + np.uint32(4))
    x0, x1 = rounds(x0, x1, _ROT0)
    x0 = x0 + _KS2
    x1 = x1 + np.uint32(_KS0 + np.uint32(5))
    return x0 ^ x1


def _ratio(p_blk, base_col, n_cols, n_total, blk_w):
    b_rows, _ = p_blk.shape
    row = jax.lax.broadcasted_iota(jnp.uint32, (b_rows, blk_w), 0)
    col_u = jax.lax.broadcasted_iota(jnp.uint32, (b_rows, blk_w), 1)
    lo = row * np.uint32(n_cols) + col_u + base_col.astype(jnp.uint32)
    bits = _threefry_bits(lo)
    fl = pltpu_bitcast_f32((bits >> np.uint32(9)) | np.uint32(0x3F800000))
    fl = fl - np.float32(1.0)
    u = jnp.maximum(_TINY, fl * (np.float32(1.0) - _TINY) + _TINY)
    e = -jnp.log(u)
    r = p_blk / e
    col_i = jax.lax.broadcasted_iota(jnp.int32, (b_rows, blk_w), 1)
    gcol = col_i + base_col
    return jnp.where(gcol < n_total, r, np.float32(-1.0)), gcol


def pltpu_bitcast_f32(x):
    return jax.lax.bitcast_convert_type(x, jnp.float32)


def _argmax_kernel(p_ref, vmax_ref, vidx_ref, *, blk_w, n_total, n_cols):
    b = pl.program_id(0)

    @pl.when(b == 0)
    def _init():
        vmax_ref[:, :] = jnp.full(vmax_ref.shape, -2.0, jnp.float32)
        vidx_ref[:, :] = jnp.zeros(vidx_ref.shape, jnp.int32)

    base = b * blk_w
    r, gcol = _ratio(p_ref[:, :], base, n_cols, n_total, blk_w)
    bm = jnp.max(r, axis=1, keepdims=True)
    is_max = r == bm
    bi = jnp.min(jnp.where(is_max, gcol, np.int32(2**31 - 1)), axis=1,
                 keepdims=True)
    cur = vmax_ref[:, :]
    upd = bm > cur
    vmax_ref[:, :] = jnp.where(upd, bm, cur)
    vidx_ref[:, :] = jnp.where(upd, bi, vidx_ref[:, :])


def _onehot_kernel(idx_ref, o_ref, *, blk_w):
    b = pl.program_id(0)
    col = jax.lax.broadcasted_iota(jnp.int32, o_ref.shape, 1) + b * blk_w
    o_ref[:, :] = jnp.where(col == idx_ref[:, :], np.float32(1.0),
                            np.float32(0.0))


@functools.partial(jax.jit, static_argnames=())
def kernel(probs):
    n_rows, n_cols = probs.shape
    blk_w = 4096
    nb = pl.cdiv(n_cols, blk_w)

    vmax, vidx = pl.pallas_call(
        functools.partial(_argmax_kernel, blk_w=blk_w, n_total=n_cols,
                          n_cols=n_cols),
        grid=(nb,),
        in_specs=[pl.BlockSpec((n_rows, blk_w), lambda b: (0, b))],
        out_specs=[pl.BlockSpec((n_rows, 1), lambda b: (0, 0)),
                   pl.BlockSpec((n_rows, 1), lambda b: (0, 0))],
        out_shape=[jax.ShapeDtypeStruct((n_rows, 1), jnp.float32),
                   jax.ShapeDtypeStruct((n_rows, 1), jnp.int32)],
    )(probs)
    del vmax

    blk_w2 = 8192
    nb2 = pl.cdiv(n_cols, blk_w2)
    out = pl.pallas_call(
        functools.partial(_onehot_kernel, blk_w=blk_w2),
        grid=(nb2,),
        in_specs=[pl.BlockSpec((n_rows, 1), lambda b: (0, 0))],
        out_specs=pl.BlockSpec((n_rows, blk_w2), lambda b: (0, b)),
        out_shape=jax.ShapeDtypeStruct((n_rows, n_cols), jnp.float32),
    )(vidx)
    return out


# speedup vs baseline: 1.1435x; 1.1435x over previous
import functools

import jax
import jax.numpy as jnp
import numpy as np
from jax.experimental import pallas as pl

_KS0 = np.uint32(0)
_KS1 = np.uint32(42)
_KS2 = np.uint32(0x1BD11BDA ^ 42)
_ROT0 = (13, 15, 26, 6)
_ROT1 = (17, 29, 16, 24)
_TINY = np.float32(np.finfo(np.float32).tiny)


def _rotl(x, d):
    return (x << np.uint32(d)) | (x >> np.uint32(32 - d))


def _threefry_bits(x1):
    def rounds(x0, x1, rots):
        for r in rots:
            x0 = x0 + x1
            x1 = _rotl(x1, r)
            x1 = x1 ^ x0
        return x0, x1

    x0 = x1
    x1 = _rotl(x1, _ROT0[0]) ^ x0
    x0, x1 = rounds(x0, x1, _ROT0[1:])
    x0 = x0 + _KS1
    x1 = x1 + np.uint32(_KS2 + np.uint32(1))
    x0, x1 = rounds(x0, x1, _ROT1)
    x0 = x0 + _KS2
    x1 = x1 + np.uint32(2)
    x0, x1 = rounds(x0, x1, _ROT0)
    x1 = x1 + np.uint32(_KS1 + np.uint32(3))
    x0, x1 = rounds(x0, x1, _ROT1)
    x0 = x0 + _KS1
    x1 = x1 + np.uint32(_KS2 + np.uint32(4))
    x0, x1 = rounds(x0, x1, _ROT0)
    x0 = x0 + _KS2
    x1 = x1 + np.uint32(5)
    return x0 ^ x1


def pltpu_bitcast_f32(x):
    return jax.lax.bitcast_convert_type(x, jnp.float32)


def _strip_ratio(p_s, ciota_u, ciota_i, rowbase_u, base_col, n_total):
    gcol = ciota_i + base_col
    lo = rowbase_u + gcol.astype(jnp.uint32)
    bits = _threefry_bits(lo)
    fl = pltpu_bitcast_f32((bits >> np.uint32(9)) | np.uint32(0x3F800000))
    fl = fl - np.float32(1.0)
    u = jnp.maximum(_TINY, fl + _TINY)
    t = -jnp.log2(u)
    r = p_s / t
    return jnp.where(gcol < n_total, r, np.float32(-1.0)), gcol


def _argmax_kernel(p_ref, vmax_ref, vidx_ref, *, blk_w, strip_w, n_total,
                   n_cols):
    b = pl.program_id(0)
    b_rows = p_ref.shape[0]

    ciota_u = jax.lax.broadcasted_iota(jnp.uint32, (b_rows, strip_w), 1)
    ciota_i = jax.lax.broadcasted_iota(jnp.int32, (b_rows, strip_w), 1)
    row_u = jax.lax.broadcasted_iota(jnp.uint32, (b_rows, strip_w), 0)
    rowbase_u = row_u * np.uint32(n_cols) + np.uint32(42)

    base = b * blk_w
    rm = None
    ri = None
    for s in range(blk_w // strip_w):
        p_s = p_ref[:, s * strip_w:(s + 1) * strip_w]
        r, gcol = _strip_ratio(p_s, ciota_u, ciota_i, rowbase_u,
                               base + s * strip_w, n_total)
        bm = jnp.max(r, axis=1, keepdims=True)
        bi = jnp.min(jnp.where(r == bm, gcol, np.int32(2**31 - 1)), axis=1,
                     keepdims=True)
        if rm is None:
            rm, ri = bm, bi
        else:
            upd = bm > rm
            rm = jnp.where(upd, bm, rm)
            ri = jnp.where(upd, bi, ri)

    @pl.when(b == 0)
    def _init():
        vmax_ref[:, :] = rm
        vidx_ref[:, :] = ri

    @pl.when(b > 0)
    def _acc():
        cur = vmax_ref[:, :]
        upd = rm > cur
        vmax_ref[:, :] = jnp.where(upd, rm, cur)
        vidx_ref[:, :] = jnp.where(upd, ri, vidx_ref[:, :])


def _onehot_kernel(idx_ref, o_ref, *, blk_w):
    b = pl.program_id(0)
    col = jax.lax.broadcasted_iota(jnp.int32, o_ref.shape, 1) + b * blk_w
    o_ref[:, :] = jnp.where(col == idx_ref[:, :], np.float32(1.0),
                            np.float32(0.0))


@functools.partial(jax.jit, static_argnames=())
def kernel(probs):
    n_rows, n_cols = probs.shape
    blk_w = 4096
    nb = pl.cdiv(n_cols, blk_w)

    vmax, vidx = pl.pallas_call(
        functools.partial(_argmax_kernel, blk_w=blk_w, strip_w=512,
                          n_total=n_cols, n_cols=n_cols),
        grid=(nb,),
        in_specs=[pl.BlockSpec((n_rows, blk_w), lambda b: (0, b))],
        out_specs=[pl.BlockSpec((n_rows, 1), lambda b: (0, 0)),
                   pl.BlockSpec((n_rows, 1), lambda b: (0, 0))],
        out_shape=[jax.ShapeDtypeStruct((n_rows, 1), jnp.float32),
                   jax.ShapeDtypeStruct((n_rows, 1), jnp.int32)],
    )(probs)
    del vmax

    blk_w2 = 8192
    nb2 = pl.cdiv(n_cols, blk_w2)
    out = pl.pallas_call(
        functools.partial(_onehot_kernel, blk_w=blk_w2),
        grid=(nb2,),
        in_specs=[pl.BlockSpec((n_rows, 1), lambda b: (0, 0))],
        out_specs=pl.BlockSpec((n_rows, blk_w2), lambda b: (0, b)),
        out_shape=jax.ShapeDtypeStruct((n_rows, n_cols), jnp.float32),
    )(vidx)
    return out


# speedup vs baseline: 1.1860x; 1.0372x over previous
import functools

import jax
import jax.numpy as jnp
import numpy as np
from jax.experimental import pallas as pl

_KS0 = np.uint32(0)
_KS1 = np.uint32(42)
_KS2 = np.uint32(0x1BD11BDA ^ 42)
_ROT0 = (13, 15, 26, 6)
_ROT1 = (17, 29, 16, 24)
_TINY = np.float32(np.finfo(np.float32).tiny)


def _rotl(x, d):
    return (x << np.uint32(d)) | (x >> np.uint32(32 - d))


def _threefry_bits(x1):
    def rounds(x0, x1, rots):
        for r in rots:
            x0 = x0 + x1
            x1 = _rotl(x1, r)
            x1 = x1 ^ x0
        return x0, x1

    x0 = x1
    x1 = _rotl(x1, _ROT0[0]) ^ x0
    x0, x1 = rounds(x0, x1, _ROT0[1:])
    x0 = x0 + _KS1
    x1 = x1 + np.uint32(_KS2 + np.uint32(1))
    x0, x1 = rounds(x0, x1, _ROT1)
    x0 = x0 + _KS2
    x1 = x1 + np.uint32(2)
    x0, x1 = rounds(x0, x1, _ROT0)
    x1 = x1 + np.uint32(_KS1 + np.uint32(3))
    x0, x1 = rounds(x0, x1, _ROT1)
    x0 = x0 + _KS1
    x1 = x1 + np.uint32(_KS2 + np.uint32(4))
    x0, x1 = rounds(x0, x1, _ROT0)
    x0 = x0 + _KS2
    x1 = x1 + np.uint32(5)
    return x0 ^ x1


def pltpu_bitcast_f32(x):
    return jax.lax.bitcast_convert_type(x, jnp.float32)


def _strip_ratio(p_s, ciota_u, ciota_i, rowbase_u, base_col, n_total):
    gcol = ciota_i + base_col
    lo = rowbase_u + gcol.astype(jnp.uint32)
    bits = _threefry_bits(lo)
    fl = pltpu_bitcast_f32((bits >> np.uint32(9)) | np.uint32(0x3F800000))
    fl = fl - np.float32(1.0)
    u = jnp.maximum(_TINY, fl + _TINY)
    t = -jnp.log2(u)
    r = p_s / t
    return jnp.where(gcol < n_total, r, np.float32(-1.0)), gcol


def _argmax_kernel(p_ref, vmax_ref, vidx_ref, *, blk_w, strip_w, n_total,
                   n_cols):
    b = pl.program_id(0)
    b_rows = p_ref.shape[0]

    ciota_u = jax.lax.broadcasted_iota(jnp.uint32, (b_rows, strip_w), 1)
    ciota_i = jax.lax.broadcasted_iota(jnp.int32, (b_rows, strip_w), 1)
    row_u = jax.lax.broadcasted_iota(jnp.uint32, (b_rows, strip_w), 0)
    rowbase_u = row_u * np.uint32(n_cols) + np.uint32(42)

    base = b * blk_w
    rm = None
    ri = None
    for s in range(blk_w // strip_w):
        p_s = p_ref[:, s * strip_w:(s + 1) * strip_w]
        r, gcol = _strip_ratio(p_s, ciota_u, ciota_i, rowbase_u,
                               base + s * strip_w, n_total)
        bm = jnp.max(r, axis=1, keepdims=True)
        bi = jnp.min(jnp.where(r == bm, gcol, np.int32(2**31 - 1)), axis=1,
                     keepdims=True)
        if rm is None:
            rm, ri = bm, bi
        else:
            upd = bm > rm
            rm = jnp.where(upd, bm, rm)
            ri = jnp.where(upd, bi, ri)

    @pl.when(b == 0)
    def _init():
        vmax_ref[:, :] = rm
        vidx_ref[:, :] = ri

    @pl.when(b > 0)
    def _acc():
        cur = vmax_ref[:, :]
        upd = rm > cur
        vmax_ref[:, :] = jnp.where(upd, rm, cur)
        vidx_ref[:, :] = jnp.where(upd, ri, vidx_ref[:, :])


def _onehot_kernel(idx_ref, o_ref, *, blk_w):
    b = pl.program_id(0)
    col = jax.lax.broadcasted_iota(jnp.int32, o_ref.shape, 1) + b * blk_w
    o_ref[:, :] = jnp.where(col == idx_ref[:, :], np.float32(1.0),
                            np.float32(0.0))


@functools.partial(jax.jit, static_argnames=())
def kernel(probs):
    n_rows, n_cols = probs.shape
    blk_w = 8192
    nb = pl.cdiv(n_cols, blk_w)

    vmax, vidx = pl.pallas_call(
        functools.partial(_argmax_kernel, blk_w=blk_w, strip_w=512,
                          n_total=n_cols, n_cols=n_cols),
        grid=(nb,),
        in_specs=[pl.BlockSpec((n_rows, blk_w), lambda b: (0, b))],
        out_specs=[pl.BlockSpec((n_rows, 1), lambda b: (0, 0)),
                   pl.BlockSpec((n_rows, 1), lambda b: (0, 0))],
        out_shape=[jax.ShapeDtypeStruct((n_rows, 1), jnp.float32),
                   jax.ShapeDtypeStruct((n_rows, 1), jnp.int32)],
    )(probs)
    del vmax

    blk_w2 = 8192
    nb2 = pl.cdiv(n_cols, blk_w2)
    out = pl.pallas_call(
        functools.partial(_onehot_kernel, blk_w=blk_w2),
        grid=(nb2,),
        in_specs=[pl.BlockSpec((n_rows, 1), lambda b: (0, 0))],
        out_specs=pl.BlockSpec((n_rows, blk_w2), lambda b: (0, b)),
        out_shape=jax.ShapeDtypeStruct((n_rows, n_cols), jnp.float32),
    )(vidx)
    return out
